# Initial kernel scaffold; baseline (speedup 1.0000x reference)
import jax
import jax.numpy as jnp
from jax.experimental import pallas as pl


def kernel(class_logits, boxes_preds, class_labels, boxes_labels, image_size):
    raise NotImplementedError("write your pallas kernel here")


# speedup vs baseline: 1.1058x; 1.1058x over previous
import functools

import jax
import jax.numpy as jnp
from jax import lax
from jax.experimental import pallas as pl

NUM_CLASSES = 80
ALPHA = 0.1
GAMMA = 0.2
EPS = 1e-7


def _loss_kernel(logits_ref, bpred_ref, labels_ref, btgt_ref, out_ref):
    b = pl.program_id(0)

    l = logits_ref[0]
    bp = bpred_ref[0]
    lab = labels_ref[0, 0]
    bt = btgt_ref[0]

    Q, C = l.shape
    T = lab.shape[0]

    p = jax.nn.sigmoid(l)
    sp0 = jnp.maximum(l, 0.0) + jnp.log1p(jnp.exp(-jnp.abs(l)))
    pg = p ** GAMMA
    bg_sum = jnp.sum((1.0 - ALPHA) * sp0 * pg)

    pos = ALPHA * (1.0 - p) ** GAMMA * (-jnp.log(p + EPS))
    neg = (1.0 - ALPHA) * pg * (-jnp.log(1.0 - p + EPS))
    cdiff = pos - neg
    oh_ct = (lax.broadcasted_iota(jnp.int32, (C, T), 0) == lab[None, :])
    cost_class = lax.dot_general(
        cdiff, oh_ct.astype(jnp.float32), (((1,), (0,)), ((), ())),
        precision=lax.Precision.HIGHEST, preferred_element_type=jnp.float32)

    ax1, ay1, ax2, ay2 = (bp[:, i][:, None] for i in range(4))
    bx1, by1, bx2, by2 = (bt[:, i][None, :] for i in range(4))
    cost_bbox = (jnp.abs(ax1 - bx1) + jnp.abs(ay1 - by1)
                 + jnp.abs(ax2 - bx2) + jnp.abs(ay2 - by2))
    area_a = (ax2 - ax1) * (ay2 - ay1)
    area_b = (bx2 - bx1) * (by2 - by1)
    iw = jnp.maximum(jnp.minimum(ax2, bx2) - jnp.maximum(ax1, bx1), 0.0)
    ih = jnp.maximum(jnp.minimum(ay2, by2) - jnp.maximum(ay1, by1), 0.0)
    inter = iw * ih
    union = area_a + area_b - inter
    iou = inter / (union + EPS)
    cw = jnp.maximum(jnp.maximum(ax2, bx2) - jnp.minimum(ax1, bx1), 0.0)
    ch = jnp.maximum(jnp.maximum(ay2, by2) - jnp.minimum(ay1, by1), 0.0)
    area_c = cw * ch
    giou_qt = iou - (area_c - union) / (area_c + EPS)

    cost = cost_class + cost_bbox - giou_qt

    minv = jnp.min(cost, axis=0)
    qio = lax.broadcasted_iota(jnp.int32, (Q, T), 0)
    src = jnp.min(jnp.where(cost == minv[None, :], qio, Q), axis=0)

    sel = (lax.broadcasted_iota(jnp.int32, (T, Q), 1) == src[:, None])
    self32 = sel.astype(jnp.float32)
    sel_l = lax.dot_general(
        self32, l, (((1,), (0,)), ((), ())),
        precision=lax.Precision.HIGHEST, preferred_element_type=jnp.float32)
    bp_sel = lax.dot_general(
        self32, bp, (((1,), (0,)), ((), ())),
        precision=lax.Precision.HIGHEST, preferred_element_type=jnp.float32)

    oh_tc = (lax.broadcasted_iota(jnp.int32, (T, C), 1) == lab[:, None])
    l_t = jnp.sum(jnp.where(oh_tc, sel_l, 0.0), axis=1)

    p_m = jax.nn.sigmoid(l_t)
    sp_m = jnp.maximum(l_t, 0.0) + jnp.log1p(jnp.exp(-jnp.abs(l_t)))
    fg = ALPHA * (sp_m - l_t) * (1.0 - p_m) ** GAMMA
    bg_m = (1.0 - ALPHA) * sp_m * p_m ** GAMMA
    tio = lax.broadcasted_iota(jnp.int32, (T, T), 0)
    tjo = lax.broadcasted_iota(jnp.int32, (T, T), 1)
    clobbered = jnp.any((src[:, None] == src[None, :]) & (tjo > tio), axis=1)
    corr = jnp.sum(jnp.where(clobbered, 0.0, fg - bg_m))

    sx1, sy1, sx2, sy2 = (bp_sel[:, i] for i in range(4))
    tx1, ty1, tx2, ty2 = (bt[:, i] for i in range(4))
    area_s = (sx2 - sx1) * (sy2 - sy1)
    area_t = (tx2 - tx1) * (ty2 - ty1)
    eiw = jnp.maximum(jnp.minimum(sx2, tx2) - jnp.maximum(sx1, tx1), 0.0)
    eih = jnp.maximum(jnp.minimum(sy2, ty2) - jnp.maximum(sy1, ty1), 0.0)
    einter = eiw * eih
    eunion = area_s + area_t - einter
    eiou = einter / (eunion + EPS)
    ecw = jnp.maximum(jnp.maximum(sx2, tx2) - jnp.minimum(sx1, tx1), 0.0)
    ech = jnp.maximum(jnp.maximum(sy2, ty2) - jnp.minimum(sy1, ty1), 0.0)
    earea_c = ecw * ech
    egiou = eiou - (earea_c - eunion) / (earea_c + EPS)
    giou_sum = jnp.sum(1.0 - egiou)
    bbox_sum = jnp.sum(jnp.abs(bp_sel - bt))

    sub = lax.broadcasted_iota(jnp.int32, (8, 128), 0)
    lane = lax.broadcasted_iota(jnp.int32, (8, 128), 1)
    row0 = sub == 0
    add = (jnp.where(row0 & (lane == 0), bg_sum + corr, 0.0)
           + jnp.where(row0 & (lane == 1), giou_sum, 0.0)
           + jnp.where(row0 & (lane == 2), bbox_sum, 0.0))

    @pl.when(b == 0)
    def _():
        out_ref[...] = jnp.zeros_like(out_ref)

    out_ref[...] += add


@functools.partial(jax.jit, static_argnames=("interpret",))
def kernel(class_logits, boxes_preds, class_labels, boxes_labels,
           image_size, interpret=False):
    B, Q, C = class_logits.shape
    T = class_labels.shape[1]
    labels3 = class_labels.reshape(B, 1, T)
    out = pl.pallas_call(
        _loss_kernel,
        grid=(B,),
        in_specs=[
            pl.BlockSpec((1, Q, C), lambda b: (b, 0, 0)),
            pl.BlockSpec((1, Q, 4), lambda b: (b, 0, 0)),
            pl.BlockSpec((1, 1, T), lambda b: (b, 0, 0)),
            pl.BlockSpec((1, T, 4), lambda b: (b, 0, 0)),
        ],
        out_specs=pl.BlockSpec((8, 128), lambda b: (0, 0)),
        out_shape=jax.ShapeDtypeStruct((8, 128), jnp.float32),
        interpret=interpret,
    )(class_logits, boxes_preds, labels3, boxes_labels)
    return out[0, 0], out[0, 1], out[0, 2] / image_size[0]


# speedup vs baseline: 3.0194x; 2.7305x over previous
import functools

import jax
import jax.numpy as jnp
from jax import lax
from jax.experimental import pallas as pl
from jax.experimental.pallas import tpu as pltpu

NUM_CLASSES = 80
ALPHA = 0.1
GAMMA = 0.2
EPS = 1e-7


def _loss_kernel(logits_ref, bpred_ref, labels_ref, btgt_ref, out_ref):
    l = logits_ref[0]
    bp = bpred_ref[0]
    lab = labels_ref[0, 0]
    bt = btgt_ref[0]

    C, Q = l.shape
    T = lab.shape[0]

    p = jax.nn.sigmoid(l)
    sp0 = jnp.maximum(l, 0.0) + jnp.log1p(jnp.exp(-jnp.abs(l)))
    pg = p ** GAMMA
    bg_sum = jnp.sum((1.0 - ALPHA) * sp0 * pg)

    pos = ALPHA * (1.0 - p) ** GAMMA * (-jnp.log(p + EPS))
    neg = (1.0 - ALPHA) * pg * (-jnp.log(1.0 - p + EPS))
    cdiff = pos - neg
    oh_tc = (lax.broadcasted_iota(jnp.int32, (T, C), 1) == lab[:, None])
    ohf = oh_tc.astype(jnp.float32)
    cost_class = lax.dot_general(
        ohf, cdiff, (((1,), (0,)), ((), ())),
        precision=lax.Precision.HIGHEST, preferred_element_type=jnp.float32)
    lsel = lax.dot_general(
        ohf, l, (((1,), (0,)), ((), ())),
        precision=lax.Precision.HIGHEST, preferred_element_type=jnp.float32)

    ax1, ay1, ax2, ay2 = (bp[i][None, :] for i in range(4))
    tx1, ty1, tx2, ty2 = (bt[:, i][:, None] for i in range(4))
    cost_bbox = (jnp.abs(ax1 - tx1) + jnp.abs(ay1 - ty1)
                 + jnp.abs(ax2 - tx2) + jnp.abs(ay2 - ty2))
    area_a = (ax2 - ax1) * (ay2 - ay1)
    area_t = (tx2 - tx1) * (ty2 - ty1)
    iw = jnp.maximum(jnp.minimum(ax2, tx2) - jnp.maximum(ax1, tx1), 0.0)
    ih = jnp.maximum(jnp.minimum(ay2, ty2) - jnp.maximum(ay1, ty1), 0.0)
    inter = iw * ih
    union = area_a + area_t - inter
    iou = inter / (union + EPS)
    cw = jnp.maximum(jnp.maximum(ax2, tx2) - jnp.minimum(ax1, tx1), 0.0)
    ch = jnp.maximum(jnp.maximum(ay2, ty2) - jnp.minimum(ay1, ty1), 0.0)
    area_c = cw * ch
    giou_tq = iou - (area_c - union) / (area_c + EPS)

    cost = cost_class + cost_bbox - giou_tq

    minv = jnp.min(cost, axis=1)
    qio = lax.broadcasted_iota(jnp.int32, (T, Q), 1)
    src = jnp.min(jnp.where(cost == minv[:, None], qio, Q), axis=1)

    sel = (qio == src[:, None]).astype(jnp.float32)
    l_t = jnp.sum(sel * lsel, axis=1)
    sx1 = jnp.sum(sel * bp[0][None, :], axis=1)
    sy1 = jnp.sum(sel * bp[1][None, :], axis=1)
    sx2 = jnp.sum(sel * bp[2][None, :], axis=1)
    sy2 = jnp.sum(sel * bp[3][None, :], axis=1)

    p_m = jax.nn.sigmoid(l_t)
    sp_m = jnp.maximum(l_t, 0.0) + jnp.log1p(jnp.exp(-jnp.abs(l_t)))
    fg = ALPHA * (sp_m - l_t) * (1.0 - p_m) ** GAMMA
    bg_m = (1.0 - ALPHA) * sp_m * p_m ** GAMMA
    tio = lax.broadcasted_iota(jnp.int32, (T, T), 0)
    tjo = lax.broadcasted_iota(jnp.int32, (T, T), 1)
    clobbered = jnp.any((src[:, None] == src[None, :]) & (tjo > tio), axis=1)
    corr = jnp.sum(jnp.where(clobbered, 0.0, fg - bg_m))

    ttx1, tty1, ttx2, tty2 = (bt[:, i] for i in range(4))
    area_s = (sx2 - sx1) * (sy2 - sy1)
    area_tt = (ttx2 - ttx1) * (tty2 - tty1)
    eiw = jnp.maximum(jnp.minimum(sx2, ttx2) - jnp.maximum(sx1, ttx1), 0.0)
    eih = jnp.maximum(jnp.minimum(sy2, tty2) - jnp.maximum(sy1, tty1), 0.0)
    einter = eiw * eih
    eunion = area_s + area_tt - einter
    eiou = einter / (eunion + EPS)
    ecw = jnp.maximum(jnp.maximum(sx2, ttx2) - jnp.minimum(sx1, ttx1), 0.0)
    ech = jnp.maximum(jnp.maximum(sy2, tty2) - jnp.minimum(sy1, tty1), 0.0)
    earea_c = ecw * ech
    egiou = eiou - (earea_c - eunion) / (earea_c + EPS)
    giou_sum = jnp.sum(1.0 - egiou)
    bbox_sum = (jnp.sum(jnp.abs(sx1 - ttx1)) + jnp.sum(jnp.abs(sy1 - tty1))
                + jnp.sum(jnp.abs(sx2 - ttx2)) + jnp.sum(jnp.abs(sy2 - tty2)))

    sub = lax.broadcasted_iota(jnp.int32, (8, 128), 0)
    lane = lax.broadcasted_iota(jnp.int32, (8, 128), 1)
    row0 = sub == 0
    out_ref[0] = (jnp.where(row0 & (lane == 0), bg_sum + corr, 0.0)
                  + jnp.where(row0 & (lane == 1), giou_sum, 0.0)
                  + jnp.where(row0 & (lane == 2), bbox_sum, 0.0))


@functools.partial(jax.jit, static_argnames=("interpret",))
def kernel(class_logits, boxes_preds, class_labels, boxes_labels,
           image_size, interpret=False):
    B, Q, C = class_logits.shape
    T = class_labels.shape[1]
    lT = class_logits.transpose(0, 2, 1)
    bpT = boxes_preds.transpose(0, 2, 1)
    labels3 = class_labels.reshape(B, 1, T)
    out = pl.pallas_call(
        _loss_kernel,
        grid=(B,),
        in_specs=[
            pl.BlockSpec((1, C, Q), lambda b: (b, 0, 0)),
            pl.BlockSpec((1, 4, Q), lambda b: (b, 0, 0)),
            pl.BlockSpec((1, 1, T), lambda b: (b, 0, 0)),
            pl.BlockSpec((1, T, 4), lambda b: (b, 0, 0)),
        ],
        out_specs=pl.BlockSpec((1, 8, 128), lambda b: (b, 0, 0)),
        out_shape=jax.ShapeDtypeStruct((B, 8, 128), jnp.float32),
        compiler_params=pltpu.CompilerParams(
            dimension_semantics=("parallel",)),
        interpret=interpret,
    )(lT, bpT, labels3, boxes_labels)
    sums = jnp.sum(out[:, 0, :3], axis=0)
    return sums[0], sums[1], sums[2] / image_size[0]
